# baseline (device time: 57552 ns/iter reference)
import jax
import jax.numpy as jnp
from jax import lax
from jax.experimental import pallas as pl
from jax.experimental.pallas import tpu as pltpu

N_DEV = 4
B, SQ, SKV, HQ, DH = 2, 256, 1024, 4, 64
S_LOC = SKV // N_DEV
DM = 512
BLK = 64


def kernel(x, Wq, K_ext, V_ext, Wo):
    def body(x_ref, wq_ref, k_ref, v_ref, wo_ref, out_ref,
             kg_ref, vg_ref, ksend, krecv, vsend, vrecv):
        my = lax.axis_index("i")
        left = (my + N_DEV - 1) % N_DEV
        right = (my + 1) % N_DEV

        barrier = pltpu.get_barrier_semaphore()
        for nbr in (left, right):
            pl.semaphore_signal(barrier, inc=1, device_id=(nbr,),
                                device_id_type=pl.DeviceIdType.MESH)
        pl.semaphore_wait(barrier, 2)

        kg_ref[pl.ds(my, 1)] = k_ref[...].astype(jnp.bfloat16)[None]
        vg_ref[pl.ds(my, 1)] = v_ref[...].astype(jnp.bfloat16)[None]

        for h in range(N_DEV - 1):
            origin = (my + N_DEV - h) % N_DEV
            krdma = pltpu.make_async_remote_copy(
                src_ref=kg_ref.at[origin], dst_ref=kg_ref.at[origin],
                send_sem=ksend.at[h], recv_sem=krecv.at[h],
                device_id=(right,), device_id_type=pl.DeviceIdType.MESH)
            vrdma = pltpu.make_async_remote_copy(
                src_ref=vg_ref.at[origin], dst_ref=vg_ref.at[origin],
                send_sem=vsend.at[h], recv_sem=vrecv.at[h],
                device_id=(right,), device_id_type=pl.DeviceIdType.MESH)
            krdma.start()
            vrdma.start()
            krdma.wait()
            vrdma.wait()

        wq = wq_ref[...].astype(jnp.bfloat16)
        wo = wo_ref[...].astype(jnp.bfloat16)
        qi = lax.broadcasted_iota(jnp.int32, (SQ, SKV), 0) // BLK
        kj = lax.broadcasted_iota(jnp.int32, (SQ, SKV), 1) // BLK
        mask = (qi == kj) | ((kj % 4) == (qi % 4))
        for b in range(B):
            xb = x_ref[b].astype(jnp.bfloat16)
            q = jnp.dot(xb, wq, preferred_element_type=jnp.float32)
            kb_full = jnp.concatenate(
                [kg_ref[j, b] for j in range(N_DEV)], axis=0)
            vb_full = jnp.concatenate(
                [vg_ref[j, b] for j in range(N_DEV)], axis=0)
            ctx_cols = []
            for h in range(HQ):
                qh = q[:, h * DH:(h + 1) * DH].astype(jnp.bfloat16)
                kh = kb_full[:, h, :]
                vh = vb_full[:, h, :]
                s = lax.dot_general(
                    qh, kh, (((1,), (1,)), ((), ())),
                    preferred_element_type=jnp.float32) * 0.125
                s = jnp.where(mask, s, -1e9)
                m = jnp.max(s, axis=-1, keepdims=True)
                w = jnp.exp(s - m)
                w = w / jnp.sum(w, axis=-1, keepdims=True)
                ctx_cols.append(jnp.dot(w.astype(jnp.bfloat16), vh,
                                        preferred_element_type=jnp.float32))
            ctx = jnp.concatenate(ctx_cols, axis=1)
            out_ref[b] = jnp.dot(ctx.astype(jnp.bfloat16), wo,
                                 preferred_element_type=jnp.float32)

    return pl.pallas_call(
        body,
        out_shape=jax.ShapeDtypeStruct((B, SQ, DM), jnp.float32),
        in_specs=[pl.BlockSpec(memory_space=pltpu.VMEM)] * 5,
        out_specs=pl.BlockSpec(memory_space=pltpu.VMEM),
        scratch_shapes=[
            pltpu.VMEM((N_DEV, B, S_LOC, HQ, DH), jnp.bfloat16),
            pltpu.VMEM((N_DEV, B, S_LOC, HQ, DH), jnp.bfloat16),
            pltpu.SemaphoreType.DMA((N_DEV - 1,)),
            pltpu.SemaphoreType.DMA((N_DEV - 1,)),
            pltpu.SemaphoreType.DMA((N_DEV - 1,)),
            pltpu.SemaphoreType.DMA((N_DEV - 1,)),
        ],
        compiler_params=pltpu.CompilerParams(collective_id=0),
    )(x, Wq, K_ext, V_ext, Wo)


# device time: 24803 ns/iter; 2.3204x vs baseline; 2.3204x over previous
import jax
import jax.numpy as jnp
from jax import lax
from jax.experimental import pallas as pl
from jax.experimental.pallas import tpu as pltpu

N_DEV = 4
B, SQ, SKV, HQ, DH = 2, 256, 1024, 4, 64
S_LOC = SKV // N_DEV
DM = 512
BLK = 64


def kernel(x, Wq, K_ext, V_ext, Wo):
    def body(x_ref, wq_ref, k_ref, v_ref, wo_ref, out_ref,
             ctxg, statsg, csend, crecv, ssend, srecv):
        my = lax.axis_index("i")

        barrier = pltpu.get_barrier_semaphore()
        for off in (1, 2, 3):
            pl.semaphore_signal(barrier, inc=1,
                                device_id=((my + off) % N_DEV,),
                                device_id_type=pl.DeviceIdType.MESH)
        pl.semaphore_wait(barrier, N_DEV - 1)

        wq = wq_ref[...].astype(jnp.bfloat16)
        qi = lax.broadcasted_iota(jnp.int32, (SQ, S_LOC), 0) // BLK
        kj = lax.broadcasted_iota(jnp.int32, (SQ, S_LOC), 1) // BLK
        mask_f = (qi == kj).astype(jnp.float32)

        ctx_b = []
        stats_b = []
        for b in range(B):
            xb = x_ref[b].astype(jnp.bfloat16)
            q = jnp.dot(xb, wq, preferred_element_type=jnp.float32)
            kb = k_ref[b]
            vb = v_ref[b]
            ctx_h = []
            s_h = []
            for h in range(HQ):
                qh = q[:, h * DH:(h + 1) * DH].astype(jnp.bfloat16)
                kh = kb[:, h, :].astype(jnp.bfloat16)
                vh = vb[:, h, :].astype(jnp.bfloat16)
                s = lax.dot_general(
                    qh, kh, (((1,), (1,)), ((), ())),
                    preferred_element_type=jnp.float32) * 0.125
                e = jnp.exp(s) * mask_f
                s_h.append(jnp.sum(e, axis=-1))
                ctx_h.append(jnp.dot(e.astype(jnp.bfloat16), vh,
                                     preferred_element_type=jnp.float32))
            ctx_b.append(jnp.stack(ctx_h, axis=1))
            stats_b.append(jnp.stack(s_h, axis=0))
        ctx_all = jnp.stack(ctx_b, axis=0)
        stats_all = jnp.stack(stats_b, axis=0)

        ctxg[pl.ds(my, 1)] = ctx_all.astype(jnp.bfloat16)[None]
        statsg[pl.ds(my, 1)] = stats_all[None]

        rdmas = []
        for idx, off in enumerate((1, 2, 3)):
            peer = (my + off) % N_DEV
            crdma = pltpu.make_async_remote_copy(
                src_ref=ctxg.at[my], dst_ref=ctxg.at[my],
                send_sem=csend.at[idx], recv_sem=crecv.at[idx],
                device_id=(peer,), device_id_type=pl.DeviceIdType.MESH)
            srdma = pltpu.make_async_remote_copy(
                src_ref=statsg.at[my], dst_ref=statsg.at[my],
                send_sem=ssend.at[idx], recv_sem=srecv.at[idx],
                device_id=(peer,), device_id_type=pl.DeviceIdType.MESH)
            crdma.start()
            srdma.start()
            rdmas.append((crdma, srdma))
        for crdma, srdma in rdmas:
            crdma.wait()
            srdma.wait()

        wo = wo_ref[...].astype(jnp.bfloat16)
        for b in range(B):
            denom = (statsg[0, b] + statsg[1, b]
                     + statsg[2, b] + statsg[3, b])
            num = (ctxg[0, b].astype(jnp.float32)
                   + ctxg[1, b].astype(jnp.float32)
                   + ctxg[2, b].astype(jnp.float32)
                   + ctxg[3, b].astype(jnp.float32))
            ctx = num / denom.T[:, :, None]
            out_ref[b] = jnp.dot(ctx.reshape(SQ, HQ * DH).astype(jnp.bfloat16),
                                 wo, preferred_element_type=jnp.float32)

    return pl.pallas_call(
        body,
        out_shape=jax.ShapeDtypeStruct((B, SQ, DM), jnp.float32),
        in_specs=[pl.BlockSpec(memory_space=pltpu.VMEM)] * 5,
        out_specs=pl.BlockSpec(memory_space=pltpu.VMEM),
        scratch_shapes=[
            pltpu.VMEM((N_DEV, B, SQ, HQ, DH), jnp.bfloat16),
            pltpu.VMEM((N_DEV, B, HQ, SQ), jnp.float32),
            pltpu.SemaphoreType.DMA((N_DEV - 1,)),
            pltpu.SemaphoreType.DMA((N_DEV - 1,)),
            pltpu.SemaphoreType.DMA((N_DEV - 1,)),
            pltpu.SemaphoreType.DMA((N_DEV - 1,)),
        ],
        compiler_params=pltpu.CompilerParams(collective_id=0),
    )(x, Wq, K_ext, V_ext, Wo)


# device time: 18323 ns/iter; 3.1410x vs baseline; 1.3537x over previous
import jax
import jax.numpy as jnp
from jax import lax
from jax.experimental import pallas as pl
from jax.experimental.pallas import tpu as pltpu

N_DEV = 4
B, SQ, SKV, HQ, DH = 2, 256, 1024, 4, 64
S_LOC = SKV // N_DEV
DM = 512
BLK = 64
HD = HQ * DH
ROWS = SQ + 16


def kernel(x, Wq, K_ext, V_ext, Wo):
    def body(x_ref, wq_ref, k_ref, v_ref, wo_ref, out_ref,
             sendbuf, csend, crecv):
        my = lax.axis_index("i")

        wq = wq_ref[...].astype(jnp.bfloat16)
        qi = lax.broadcasted_iota(jnp.int32, (SQ, S_LOC), 0) // BLK
        kj = lax.broadcasted_iota(jnp.int32, (SQ, S_LOC), 1) // BLK
        mask_f = (qi == kj).astype(jnp.float32)

        ctx_own = []
        stats_own = []
        for b in range(B):
            xb = x_ref[b].astype(jnp.bfloat16)
            q = jnp.dot(xb, wq, preferred_element_type=jnp.float32)
            kb = k_ref[b].reshape(S_LOC, HD).astype(jnp.bfloat16)
            vb = v_ref[b].reshape(S_LOC, HD).astype(jnp.bfloat16)
            ctx_h = []
            s_h = []
            for h in range(HQ):
                qh = q[:, h * DH:(h + 1) * DH].astype(jnp.bfloat16)
                kh = kb[:, h * DH:(h + 1) * DH]
                vh = vb[:, h * DH:(h + 1) * DH]
                s = lax.dot_general(
                    qh, kh, (((1,), (1,)), ((), ())),
                    preferred_element_type=jnp.float32) * 0.125
                e = jnp.exp(s) * mask_f
                s_h.append(jnp.sum(e, axis=-1))
                ctx_h.append(jnp.dot(e.astype(jnp.bfloat16), vh,
                                     preferred_element_type=jnp.float32))
            ctx_own.append(jnp.concatenate(ctx_h, axis=1))
            stats_own.append(jnp.stack(s_h, axis=0))
            sendbuf[pl.ds(my, 1), b, pl.ds(0, SQ)] = (
                ctx_own[b].astype(jnp.bfloat16)[None])
            sendbuf[pl.ds(my, 1), b, pl.ds(SQ, HQ)] = (
                stats_own[b].astype(jnp.bfloat16)[None])

        barrier = pltpu.get_barrier_semaphore()
        for off in (1, 2, 3):
            pl.semaphore_signal(barrier, inc=1,
                                device_id=((my + off) % N_DEV,),
                                device_id_type=pl.DeviceIdType.MESH)
        pl.semaphore_wait(barrier, N_DEV - 1)

        rdmas = []
        for idx, off in enumerate((1, 2, 3)):
            rdma = pltpu.make_async_remote_copy(
                src_ref=sendbuf.at[my], dst_ref=sendbuf.at[my],
                send_sem=csend.at[idx], recv_sem=crecv.at[idx],
                device_id=((my + off) % N_DEV,),
                device_id_type=pl.DeviceIdType.MESH)
            rdma.start()
            rdmas.append(rdma)

        num = list(ctx_own)
        den = list(stats_own)
        for idx in range(N_DEV - 1):
            rdmas[idx].wait_recv()
            slot = (my + N_DEV - 1 - idx) % N_DEV
            arr = sendbuf[pl.ds(slot, 1)]
            for b in range(B):
                num[b] = num[b] + arr[0, b, :SQ, :].astype(jnp.float32)
                den[b] = den[b] + arr[0, b, SQ:SQ + HQ, :].astype(jnp.float32)

        wo = wo_ref[...].astype(jnp.bfloat16)
        for b in range(B):
            d = jnp.broadcast_to(den[b].T[:, :, None], (SQ, HQ, DH))
            ctx = num[b] / d.reshape(SQ, HD)
            out_ref[b] = jnp.dot(ctx.astype(jnp.bfloat16), wo,
                                 preferred_element_type=jnp.float32)

        for idx in range(N_DEV - 1):
            rdmas[idx].wait_send()

    return pl.pallas_call(
        body,
        out_shape=jax.ShapeDtypeStruct((B, SQ, DM), jnp.float32),
        in_specs=[pl.BlockSpec(memory_space=pltpu.VMEM)] * 5,
        out_specs=pl.BlockSpec(memory_space=pltpu.VMEM),
        scratch_shapes=[
            pltpu.VMEM((N_DEV, B, ROWS, HD), jnp.bfloat16),
            pltpu.SemaphoreType.DMA((N_DEV - 1,)),
            pltpu.SemaphoreType.DMA((N_DEV - 1,)),
        ],
        compiler_params=pltpu.CompilerParams(collective_id=0),
    )(x, Wq, K_ext, V_ext, Wo)


# device time: 16676 ns/iter; 3.4512x vs baseline; 1.0988x over previous
import jax
import jax.numpy as jnp
from jax import lax
from jax.experimental import pallas as pl
from jax.experimental.pallas import tpu as pltpu

N_DEV = 4
B, SQ, SKV, HQ, DH = 2, 256, 1024, 4, 64
S_LOC = SKV // N_DEV
DM = 512
BLK = 64
HD = HQ * DH
ROWS = SQ + 16


def kernel(x, Wq, K_ext, V_ext, Wo):
    def body(x_ref, wq_ref, k_ref, v_ref, wo_ref, out_ref,
             sendbuf, csend, crecv):
        my = lax.axis_index("i")

        barrier = pltpu.get_barrier_semaphore()
        for off in (1, 2, 3):
            pl.semaphore_signal(barrier, inc=1,
                                device_id=((my + off) % N_DEV,),
                                device_id_type=pl.DeviceIdType.MESH)
        pl.semaphore_wait(barrier, N_DEV - 1)

        wq = wq_ref[...].astype(jnp.bfloat16)
        wo = wo_ref[...].astype(jnp.bfloat16)
        qi = lax.broadcasted_iota(jnp.int32, (SQ, S_LOC), 0) // BLK
        kj = lax.broadcasted_iota(jnp.int32, (SQ, S_LOC), 1) // BLK
        mask_f = (qi == kj).astype(jnp.float32)

        def partial_attn(b):
            xb = x_ref[b].astype(jnp.bfloat16)
            q = jnp.dot(xb, wq, preferred_element_type=jnp.float32)
            kb = k_ref[b].reshape(S_LOC, HD).astype(jnp.bfloat16)
            vb = v_ref[b].reshape(S_LOC, HD).astype(jnp.bfloat16)
            ctx_h = []
            s_h = []
            for h in range(HQ):
                qh = q[:, h * DH:(h + 1) * DH].astype(jnp.bfloat16)
                kh = kb[:, h * DH:(h + 1) * DH]
                vh = vb[:, h * DH:(h + 1) * DH]
                s = lax.dot_general(
                    qh, kh, (((1,), (1,)), ((), ())),
                    preferred_element_type=jnp.float32) * 0.125
                e = jnp.exp(s) * mask_f
                s_h.append(jnp.sum(e, axis=-1))
                ctx_h.append(jnp.dot(e.astype(jnp.bfloat16), vh,
                                     preferred_element_type=jnp.float32))
            ctx = jnp.concatenate(ctx_h, axis=1)
            stats = jnp.stack(s_h, axis=0)
            sendbuf[pl.ds(my, 1), b, pl.ds(0, SQ)] = (
                ctx.astype(jnp.bfloat16)[None])
            sendbuf[pl.ds(my, 1), b, pl.ds(SQ, HQ)] = (
                stats.astype(jnp.bfloat16)[None])
            rdmas = []
            for idx, off in enumerate((1, 2, 3)):
                rdma = pltpu.make_async_remote_copy(
                    src_ref=sendbuf.at[my, b], dst_ref=sendbuf.at[my, b],
                    send_sem=csend.at[b * 3 + idx],
                    recv_sem=crecv.at[b * 3 + idx],
                    device_id=((my + off) % N_DEV,),
                    device_id_type=pl.DeviceIdType.MESH)
                rdma.start()
                rdmas.append(rdma)
            return ctx, stats, rdmas

        def combine(b, ctx, stats, rdmas):
            num, den = ctx, stats
            for idx in range(N_DEV - 1):
                rdmas[idx].wait_recv()
                slot = (my + N_DEV - 1 - idx) % N_DEV
                arr = sendbuf[pl.ds(slot, 1), b]
                num = num + arr[0, :SQ, :].astype(jnp.float32)
                den = den + arr[0, SQ:SQ + HQ, :].astype(jnp.float32)
            d = jnp.broadcast_to(den.T[:, :, None], (SQ, HQ, DH))
            out_ref[b] = jnp.dot(
                (num / d.reshape(SQ, HD)).astype(jnp.bfloat16), wo,
                preferred_element_type=jnp.float32)

        ctx0, stats0, rdmas0 = partial_attn(0)
        ctx1, stats1, rdmas1 = partial_attn(1)
        combine(0, ctx0, stats0, rdmas0)
        combine(1, ctx1, stats1, rdmas1)

        for rdmas in (rdmas0, rdmas1):
            for idx in range(N_DEV - 1):
                rdmas[idx].wait_send()

    return pl.pallas_call(
        body,
        out_shape=jax.ShapeDtypeStruct((B, SQ, DM), jnp.float32),
        in_specs=[pl.BlockSpec(memory_space=pltpu.VMEM)] * 5,
        out_specs=pl.BlockSpec(memory_space=pltpu.VMEM),
        scratch_shapes=[
            pltpu.VMEM((N_DEV, B, ROWS, HD), jnp.bfloat16),
            pltpu.SemaphoreType.DMA((B * 3,)),
            pltpu.SemaphoreType.DMA((B * 3,)),
        ],
        compiler_params=pltpu.CompilerParams(collective_id=0),
    )(x, Wq, K_ext, V_ext, Wo)
